# baseline (speedup 1.0000x reference)
import functools

import jax
import jax.numpy as jnp
from jax import lax
from jax.experimental import pallas as pl
from jax.experimental.pallas import tpu as pltpu
from jax.experimental.pallas import tpu_sc as plsc

VOCAB = 100000
EMBED = 64
ROWS = 4096
COLS = 50
NUM_WORKERS = 32
IBLK = ROWS // NUM_WORKERS
NBUF = 5
NSTEP = COLS // NBUF

_MESH = plsc.VectorSubcoreMesh(core_axis_name="c", subcore_axis_name="s")


@functools.partial(
    pl.kernel,
    mesh=_MESH,
    out_type=jax.ShapeDtypeStruct((COLS, ROWS, EMBED), jnp.float32),
    scratch_types=[
        pltpu.VMEM((COLS, IBLK), jnp.int32),
        pltpu.VMEM((NBUF, IBLK, EMBED), jnp.float32),
        pltpu.SemaphoreType.DMA((NBUF,)),
        pltpu.SemaphoreType.DMA((NBUF,)),
    ],
    compiler_params=pltpu.CompilerParams(use_tc_tiling_on_sc=False),
)
def _sc_gather(idx_hbm, table_hbm, out_hbm, idx_v, g_v, gsem, ssem):
    wid = lax.axis_index("s") * 2 + lax.axis_index("c")
    i0 = wid * IBLK
    pltpu.sync_copy(idx_hbm.at[wid], idx_v)

    for b in range(NBUF):
        pltpu.async_copy(table_hbm.at[idx_v.at[b]], g_v.at[b], gsem.at[b])

    def superstep(s, carry):
        j0 = s * NBUF
        for b in range(NBUF):
            j = j0 + b
            pltpu.make_async_copy(
                table_hbm.at[idx_v.at[0]], g_v.at[b], gsem.at[b]).wait()
            pltpu.async_copy(
                g_v.at[b], out_hbm.at[j, pl.ds(i0, IBLK)], ssem.at[b])

            @pl.when(j + NBUF < COLS)
            def _():
                pltpu.make_async_copy(
                    g_v.at[b], out_hbm.at[0, pl.ds(i0, IBLK)],
                    ssem.at[b]).wait()
                pltpu.async_copy(
                    table_hbm.at[idx_v.at[j + NBUF]], g_v.at[b], gsem.at[b])
        return carry

    lax.fori_loop(0, NSTEP, superstep, 0)

    for b in range(NBUF):
        pltpu.make_async_copy(
            g_v.at[b], out_hbm.at[0, pl.ds(i0, IBLK)], ssem.at[b]).wait()


def _tc_transpose_body(x_ref, o_ref):
    x = x_ref[0]
    o_ref[0] = jnp.concatenate([x[:, :EMBED].T, x[:, EMBED:].T], axis=1)


_tc_transpose = pl.pallas_call(
    _tc_transpose_body,
    grid=(COLS, ROWS // 128),
    in_specs=[pl.BlockSpec((1, 64, 128), lambda j, ib: (j, ib, 0))],
    out_specs=pl.BlockSpec((1, EMBED, 128), lambda j, ib: (j, 0, ib)),
    out_shape=jax.ShapeDtypeStruct((COLS, EMBED, ROWS), jnp.float32),
)


def kernel(indices, weight):
    idx = (indices.astype(jnp.int32)
           .reshape(NUM_WORKERS, 2, IBLK // 2, COLS)
           .transpose(0, 2, 1, 3)
           .reshape(NUM_WORKERS, IBLK, COLS)
           .transpose(0, 2, 1))
    out_j = _sc_gather(idx, weight)
    xb = out_j.reshape(COLS, ROWS // 2, 2 * EMBED)
    out_t = _tc_transpose(xb)
    return out_t.transpose(2, 0, 1)
